# baseline (speedup 1.0000x reference)
import functools

import numpy as np

import jax
import jax.numpy as jnp
from jax import lax
from jax.experimental import pallas as pl
from jax.experimental.pallas import tpu as pltpu
from jax.experimental.pallas import tpu_sc as plsc

_N_TOK = 8192
_D = 2048
_MW = 8
_NC = 8
_C = _MW * _NC
_BT = 1024
_NBLK = _N_TOK // _BT

_const_cache = []


def _consts():
    if not _const_cache:
        g = jax.random.gumbel(jax.random.key(42), (_N_TOK * _MW, _NC), jnp.float32)
        noise_t = g.reshape(_N_TOK, _C).T
        _const_cache.append((jax.device_put(noise_t),
                             jnp.eye(_C, dtype=jnp.float32)))
    return _const_cache[0]


def _enc_body(x_ref, we_ref, be_ref, nt_ref, eye_ref, logits_ref, y_ref):
    lgT = jax.lax.dot_general(we_ref[...], x_ref[...], (((0,), (1,)), ((), ())),
                              preferred_element_type=jnp.float32) + be_ref[...]
    logits_ref[...] = jax.lax.dot_general(lgT, eye_ref[...],
                                          (((0,), (0,)), ((), ())),
                                          preferred_element_type=jnp.float32)
    y_ref[...] = lgT + nt_ref[...]


def _dec_body(zt_ref, eye_ref, wd_ref, bd_ref, z_ref, recon_ref):
    cdim = (((0,), (0,)), ((), ()))
    zT = zt_ref[...]
    z_ref[...] = jax.lax.dot_general(zT, eye_ref[...], cdim,
                                     preferred_element_type=jnp.float32)
    recon_ref[...] = jax.lax.dot_general(zT, wd_ref[...], cdim,
                                         preferred_element_type=jnp.float32) + bd_ref[...]


def _sc_sample(yT):
    info = plsc.get_sparse_core_info()
    nw = info.num_cores * info.num_subcores
    tpw = _N_TOK // nw
    nchunk = tpw // 16
    mesh = plsc.VectorSubcoreMesh(core_axis_name="c", subcore_axis_name="s")

    @functools.partial(
        pl.kernel, mesh=mesh,
        out_type=jax.ShapeDtypeStruct((_C, _N_TOK), jnp.float32),
        scratch_types=[
            pltpu.VMEM((_C, tpw), jnp.float32),
            pltpu.VMEM((_C, tpw), jnp.float32),
        ],
    )
    def k(y_hbm, z_hbm, y_v, z_v):
        wid = lax.axis_index("s") * info.num_cores + lax.axis_index("c")
        base = wid * tpw
        pltpu.sync_copy(y_hbm.at[:, pl.ds(base, tpw)], y_v)

        def chunk(t, _):
            sl = pl.ds(t * 16, 16)
            one = jnp.full((16,), 1.0, jnp.float32)
            zero = jnp.full((16,), 0.0, jnp.float32)
            for g in range(_MW):
                m = y_v[g * _NC, sl]
                for j in range(1, _NC):
                    m = jnp.maximum(m, y_v[g * _NC + j, sl])
                taken = zero
                for j in range(_NC):
                    v = y_v[g * _NC + j, sl]
                    hit = jnp.where(v == m, one, zero)
                    gate = jnp.where(taken < 0.5, one, zero)
                    sel = hit * gate
                    z_v[g * _NC + j, sl] = sel
                    taken = taken + sel
            return 0

        lax.fori_loop(0, nchunk, chunk, 0)
        pltpu.sync_copy(z_v, z_hbm.at[:, pl.ds(base, tpw)])

    return k(yT)


def kernel(x, W_enc, b_enc, W_dec, b_dec):
    noise_t, eye = _consts()
    full = lambda i: (0, 0)
    row = lambda i: (i, 0)
    col = lambda i: (0, i)
    logits2d, yT = pl.pallas_call(
        _enc_body,
        grid=(_NBLK,),
        in_specs=[
            pl.BlockSpec((_BT, _D), row),
            pl.BlockSpec((_D, _C), full),
            pl.BlockSpec((_C, 1), full),
            pl.BlockSpec((_C, _BT), col),
            pl.BlockSpec((_C, _C), full),
        ],
        out_specs=[
            pl.BlockSpec((_BT, _C), row),
            pl.BlockSpec((_C, _BT), col),
        ],
        out_shape=[
            jax.ShapeDtypeStruct((_N_TOK, _C), jnp.float32),
            jax.ShapeDtypeStruct((_C, _N_TOK), jnp.float32),
        ],
    )(x, W_enc, b_enc.reshape(-1, 1), noise_t, eye)
    zT = _sc_sample(yT)
    z_flat, recon = pl.pallas_call(
        _dec_body,
        grid=(_NBLK,),
        in_specs=[
            pl.BlockSpec((_C, _BT), col),
            pl.BlockSpec((_C, _C), full),
            pl.BlockSpec((_C, _D), full),
            pl.BlockSpec((1, _D), full),
        ],
        out_specs=[
            pl.BlockSpec((_BT, _C), row),
            pl.BlockSpec((_BT, _D), row),
        ],
        out_shape=[
            jax.ShapeDtypeStruct((_N_TOK, _C), jnp.float32),
            jax.ShapeDtypeStruct((_N_TOK, _D), jnp.float32),
        ],
    )(zT, eye, W_dec, b_dec.reshape(1, -1))
    return (logits2d.reshape(_N_TOK, _MW, _NC), z_flat, recon)


# speedup vs baseline: 1.3769x; 1.3769x over previous
import numpy as np

import jax
import jax.numpy as jnp
from jax.experimental import pallas as pl
from jax.experimental.pallas import tpu as pltpu

_N_TOK = 8192
_D = 2048
_MW = 8
_NC = 8
_C = _MW * _NC
_BT = 1024
_NBLK = _N_TOK // _BT

_const_cache = []


def _consts():
    if not _const_cache:
        g = jax.random.gumbel(jax.random.key(42), (_N_TOK * _MW, _NC), jnp.float32)
        noise_t = g.reshape(_N_TOK, _C).T
        c = np.arange(_C)
        grp = c // _NC
        lowt = ((grp[:, None] == grp[None, :]) & (c[None, :] < c[:, None]))
        _const_cache.append((jax.device_put(noise_t),
                             jnp.asarray(lowt.astype(np.float32)),
                             jnp.eye(_C, dtype=jnp.float32)))
    return _const_cache[0]


def _body(x_ref, we_ref, be_ref, nt_ref, lowt_ref, eye_ref, wd_ref, bd_ref,
          logits_ref, z_ref, recon_ref):
    cdim = (((0,), (0,)), ((), ()))
    lgT = jax.lax.dot_general(we_ref[...], x_ref[...], (((0,), (1,)), ((), ())),
                              preferred_element_type=jnp.float32) + be_ref[...]
    logits_ref[...] = jax.lax.dot_general(lgT, eye_ref[...], cdim,
                                          preferred_element_type=jnp.float32)
    y = (lgT + nt_ref[...]).reshape(_MW, _NC, -1)
    m = jnp.max(y, axis=1, keepdims=True)
    f = (y == m).astype(jnp.float32).reshape(_C, -1)
    dup = jax.lax.dot_general(lowt_ref[...], f, (((1,), (0,)), ((), ())),
                              preferred_element_type=jnp.float32)
    zT = jnp.where(dup == 0.0, f, 0.0)
    z_ref[...] = jax.lax.dot_general(zT, eye_ref[...], cdim,
                                     preferred_element_type=jnp.float32)
    recon_ref[...] = jax.lax.dot_general(zT, wd_ref[...], cdim,
                                         preferred_element_type=jnp.float32) + bd_ref[...]


def kernel(x, W_enc, b_enc, W_dec, b_dec):
    noise_t, lowt, eye = _consts()
    full = lambda i: (0, 0)
    row = lambda i: (i, 0)
    col = lambda i: (0, i)
    out = pl.pallas_call(
        _body,
        grid=(_NBLK,),
        in_specs=[
            pl.BlockSpec((_BT, _D), row),
            pl.BlockSpec((_D, _C), full),
            pl.BlockSpec((_C, 1), full),
            pl.BlockSpec((_C, _BT), col),
            pl.BlockSpec((_C, _C), full),
            pl.BlockSpec((_C, _C), full),
            pl.BlockSpec((_C, _D), full),
            pl.BlockSpec((1, _D), full),
        ],
        out_specs=[
            pl.BlockSpec((_BT, _C), row),
            pl.BlockSpec((_BT, _C), row),
            pl.BlockSpec((_BT, _D), row),
        ],
        out_shape=[
            jax.ShapeDtypeStruct((_N_TOK, _C), jnp.float32),
            jax.ShapeDtypeStruct((_N_TOK, _C), jnp.float32),
            jax.ShapeDtypeStruct((_N_TOK, _D), jnp.float32),
        ],
    )(x, W_enc, b_enc.reshape(-1, 1), noise_t, lowt, eye,
      W_dec, b_dec.reshape(1, -1))
    logits2d, z_flat, recon = out
    return (logits2d.reshape(_N_TOK, _MW, _NC), z_flat, recon)
